# baseline (speedup 1.0000x reference)
import functools

import jax
import jax.numpy as jnp
from jax import lax
from jax.experimental import pallas as pl
from jax.experimental.pallas import tpu as pltpu
from jax.experimental.pallas import tpu_sc as plsc

N = 10000
E = 320000
H = 128
G = 16

NC = 2
NS = 16
NW = NC * NS
B = 64
EPW = E // NW
EPWP = 10240
EBP = EPWP // B
HEB = EBP // 2
NG = HEB // 2
DW = 4
NP2 = 10240
RPT = NP2 // NS

_mesh = plsc.VectorSubcoreMesh(core_axis_name="c", subcore_axis_name="s")


@functools.partial(
    pl.kernel,
    out_type=jax.ShapeDtypeStruct((NC, NP2, H), jnp.float32),
    mesh=_mesh,
    scratch_types=[
        pltpu.VMEM((EBP, B), jnp.int32),
        pltpu.VMEM((B, H), jnp.float32),
        pltpu.VMEM_SHARED((NP2, H), jnp.float32),
        pltpu.SemaphoreType.DMA,
    ],
)
def _sc_degree(dst_hbm, ones_hbm, zeros_hbm, zblk_hbm, out_hbm, dst_v, ones_v, acc, ssem):
    c = lax.axis_index("c")
    s = lax.axis_index("s")
    wid = c * NS + s
    pltpu.sync_copy(ones_hbm, ones_v)
    row0 = s * RPT
    pltpu.sync_copy(zeros_hbm, acc.at[pl.ds(row0, RPT)])
    pltpu.sync_copy(dst_hbm.at[wid], dst_v)
    plsc.subcore_barrier()

    def wait_s():
        pltpu.make_async_copy(zblk_hbm, ones_v, ssem).wait()

    def body(i, carry):
        @pl.when(i >= DW)
        def _():
            wait_s()

        pltpu.async_copy(ones_v, acc.at[dst_v.at[i]], ssem, add=True)
        return carry

    lax.fori_loop(0, EBP, body, 0)
    for _ in range(DW):
        wait_s()
    plsc.subcore_barrier()
    pltpu.sync_copy(acc.at[pl.ds(row0, RPT)], out_hbm.at[c, pl.ds(row0, RPT)])


@functools.partial(
    pl.kernel,
    out_type=jax.ShapeDtypeStruct((NC, NP2, H), jnp.float32),
    mesh=_mesh,
    scratch_types=[
        pltpu.VMEM((HEB, B), jnp.int32),
        pltpu.VMEM((HEB, B), jnp.int32),
        pltpu.VMEM((B, H), jnp.float32),
        pltpu.VMEM((B, H), jnp.float32),
        pltpu.VMEM_SHARED((NP2, H), jnp.float32),
        pltpu.SemaphoreType.DMA,
        pltpu.SemaphoreType.DMA,
    ],
)
def _sc_scatter(table_hbm, src_hbm, dst_hbm, zeros_hbm, zblk_hbm, out_hbm,
                src_v, dst_v, rows0, rows1, acc, gsem, ssem):
    c = lax.axis_index("c")
    s = lax.axis_index("s")
    wid = c * NS + s
    row0 = s * RPT
    pltpu.sync_copy(zeros_hbm, acc.at[pl.ds(row0, RPT)])

    def gather(i, buf):
        pltpu.async_copy(table_hbm.at[src_v.at[i]], buf, gsem)

    def wait_g(buf):
        pltpu.make_async_copy(zblk_hbm, buf, gsem).wait()

    def scat(i, buf):
        pltpu.sync_copy(buf, acc.at[dst_v.at[i]], add=True)

    for half in range(2):
        pltpu.sync_copy(src_hbm.at[wid, pl.ds(half * HEB, HEB)], src_v)
        pltpu.sync_copy(dst_hbm.at[wid, pl.ds(half * HEB, HEB)], dst_v)
        if half == 0:
            plsc.subcore_barrier()
        def blk(i, carry):
            gather(i, rows0)
            wait_g(rows0)
            scat(i, rows0)
            return carry

        lax.fori_loop(0, HEB, blk, 0)
    plsc.subcore_barrier()
    pltpu.sync_copy(acc.at[pl.ds(row0, RPT)], out_hbm.at[c, pl.ds(row0, RPT)])


BN = 1000
NBLK = N // BN


def _dinv(deg_ref):
    d = deg_ref[0, :, 0:1] + deg_ref[1, :, 0:1] + 1.0
    return lax.rsqrt(jnp.maximum(d, 1.0))


def _tc_prologue_body(deg_ref, x_ref, w_ref, out_ref):
    out_ref[...] = jnp.dot(x_ref[...], w_ref[...],
                           preferred_element_type=jnp.float32) * _dinv(deg_ref)


def _combine(deg_ref, s2_ref, hwp_ref, b_ref):
    dinv = _dinv(deg_ref)
    return (s2_ref[0] + s2_ref[1] + hwp_ref[...]) * dinv + b_ref[...], dinv


def _layernorm(t, g_ref, be_ref):
    mu = jnp.mean(t, axis=-1, keepdims=True)
    var = jnp.mean((t - mu) ** 2, axis=-1, keepdims=True)
    return (t - mu) * lax.rsqrt(var + 1e-5) * g_ref[...] + be_ref[...]


def _tc_layer0_body(deg_ref, s2_ref, hwp_ref, b_ref, w_ref, h_ref, hn_ref):
    t, dinv = _combine(deg_ref, s2_ref, hwp_ref, b_ref)
    h = jnp.maximum(t, 0.0)
    h_ref[...] = h
    hn_ref[...] = jnp.dot(h, w_ref[...], preferred_element_type=jnp.float32) * dinv


def _tc_mid_body(deg_ref, s2_ref, hwp_ref, hp_ref, g_ref, be_ref, b_ref, w_ref,
                 h_ref, hn_ref):
    t, dinv = _combine(deg_ref, s2_ref, hwp_ref, b_ref)
    h = jnp.maximum(_layernorm(t, g_ref, be_ref), 0.0) + hp_ref[...]
    h_ref[...] = h
    hn_ref[...] = jnp.dot(h, w_ref[...], preferred_element_type=jnp.float32) * dinv


def _tc_final_body(deg_ref, s2_ref, hwp_ref, hp_ref, g_ref, be_ref, b_ref,
                   batch_ref, fc1w_ref, fc1b_ref, fc2w_ref, fc2b_ref,
                   out_ref, acc, cnt):
    i = pl.program_id(0)
    t, _ = _combine(deg_ref, s2_ref, hwp_ref, b_ref)
    h = jnp.maximum(_layernorm(t, g_ref, be_ref), 0.0) + hp_ref[...]

    bt = batch_ref[0, 0, :]
    onehot = (bt[:, None] == lax.broadcasted_iota(jnp.int32, (1, G), 1)
              ).astype(jnp.float32)
    part = lax.dot_general(onehot, h, (((0,), (0,)), ((), ())),
                           preferred_element_type=jnp.float32)
    cpart = lax.dot_general(onehot, jnp.ones_like(h), (((0,), (0,)), ((), ())),
                            preferred_element_type=jnp.float32)

    @pl.when(i == 0)
    def _():
        acc[...] = jnp.zeros_like(acc)
        cnt[...] = jnp.zeros_like(cnt)

    acc[...] += part
    cnt[...] += cpart

    @pl.when(i == pl.num_programs(0) - 1)
    def _():
        pooled = acc[...] / jnp.maximum(cnt[...], 1.0)
        z = jnp.maximum(jnp.dot(pooled, fc1w_ref[...],
                                preferred_element_type=jnp.float32)
                        + fc1b_ref[...], 0.0)
        o = jnp.dot(z, fc2w_ref[...], preferred_element_type=jnp.float32) \
            + fc2b_ref[...]
        out_ref[...] = 1.0 / (1.0 + jnp.exp(-o))


_deg_spec = pl.BlockSpec((2, BN, H), lambda i: (0, i, 0))
_s2_spec = pl.BlockSpec((2, BN, H), lambda i: (0, i, 0))
_row_spec = pl.BlockSpec((BN, H), lambda i: (i, 0))
_vec_spec = pl.BlockSpec((1, H), lambda i: (0, 0))
_w_spec = pl.BlockSpec((H, H), lambda i: (0, 0))

_rowout = jax.ShapeDtypeStruct((N, H), jnp.float32)

_tc_prologue = pl.pallas_call(
    _tc_prologue_body, grid=(NBLK,),
    in_specs=[_deg_spec, _row_spec, _w_spec],
    out_specs=_row_spec, out_shape=_rowout)

_tc_layer0 = pl.pallas_call(
    _tc_layer0_body, grid=(NBLK,),
    in_specs=[_deg_spec, _s2_spec, _row_spec, _vec_spec, _w_spec],
    out_specs=(_row_spec, _row_spec), out_shape=(_rowout, _rowout))

_tc_mid = pl.pallas_call(
    _tc_mid_body, grid=(NBLK,),
    in_specs=[_deg_spec, _s2_spec, _row_spec, _row_spec, _vec_spec, _vec_spec,
              _vec_spec, _w_spec],
    out_specs=(_row_spec, _row_spec), out_shape=(_rowout, _rowout))

_tc_final = pl.pallas_call(
    _tc_final_body, grid=(NBLK,),
    in_specs=[_deg_spec, _s2_spec, _row_spec, _row_spec, _vec_spec, _vec_spec,
              _vec_spec,
              pl.BlockSpec((1, 1, BN), lambda i: (i, 0, 0)),
              pl.BlockSpec((H, 64), lambda i: (0, 0)),
              pl.BlockSpec((1, 64), lambda i: (0, 0)),
              pl.BlockSpec((64, H), lambda i: (0, 0)),
              _vec_spec],
    out_specs=pl.BlockSpec((G, H), lambda i: (0, 0)),
    out_shape=jax.ShapeDtypeStruct((G, H), jnp.float32),
    scratch_shapes=[pltpu.VMEM((G, H), jnp.float32),
                    pltpu.VMEM((G, H), jnp.float32)])


def kernel(x, edge_index, batch, Ws, bs, gammas, betas, fc1_W, fc1_b, fc2_W,
           fc2_b):
    pad = EPWP - EPW
    src = jnp.pad(edge_index[0].astype(jnp.int32).reshape(NW, EPW),
                  ((0, 0), (0, pad))).reshape(NW, EBP, B)
    padrows = jnp.broadcast_to(N + jnp.arange(pad, dtype=jnp.int32) % (NP2 - N),
                               (NW, pad))
    dst = jnp.concatenate(
        [edge_index[1].astype(jnp.int32).reshape(NW, EPW), padrows],
        axis=1).reshape(NW, EBP, B)
    batch3 = batch.astype(jnp.int32).reshape(NBLK, 1, BN)

    ones16 = jnp.ones((B, H), jnp.float32)
    zerosH = jnp.zeros((RPT, H), jnp.float32)
    zblk = jnp.zeros((B, H), jnp.float32)

    bs2 = bs.reshape(8, 1, H)
    g2 = gammas.reshape(7, 1, H)
    be2 = betas.reshape(7, 1, H)
    fc1b2 = fc1_b.reshape(1, 64)
    fc2wp = jnp.pad(fc2_W, ((0, 0), (0, H - 1)))
    fc2bp = jnp.pad(fc2_b, (0, H - 1)).reshape(1, H)

    deg2 = _sc_degree(dst, ones16, zerosH, zblk)
    hwp = _tc_prologue(deg2, x, Ws[0])

    h = None
    out = None
    for i in range(8):
        s2 = _sc_scatter(hwp, src, dst, zerosH, zblk)
        if i == 0:
            h, hwp = _tc_layer0(deg2, s2, hwp, bs2[0], Ws[1])
        elif i < 7:
            h, hwp = _tc_mid(deg2, s2, hwp, h, g2[i - 1], be2[i - 1], bs2[i],
                             Ws[i + 1])
        else:
            out = _tc_final(deg2, s2, hwp, h, g2[6], be2[6], bs2[7], batch3,
                            fc1_W, fc1b2, fc2wp, fc2bp)
    return out[:, :1]


# speedup vs baseline: 1.0001x; 1.0001x over previous
import functools

import jax
import jax.numpy as jnp
from jax import lax
from jax.experimental import pallas as pl
from jax.experimental.pallas import tpu as pltpu
from jax.experimental.pallas import tpu_sc as plsc

N = 10000
E = 320000
H = 128
G = 16

NC = 2
NS = 16
NW = NC * NS
B = 64
EPW = E // NW
EPWP = 10240
EBP = EPWP // B
HEB = EBP // 2
NG = HEB // 2
DW = 4
NP2 = 10240
RPT = NP2 // NS

_mesh = plsc.VectorSubcoreMesh(core_axis_name="c", subcore_axis_name="s")


@functools.partial(
    pl.kernel,
    out_type=jax.ShapeDtypeStruct((NC, NP2, H), jnp.float32),
    mesh=_mesh,
    scratch_types=[
        pltpu.VMEM((EBP, B), jnp.int32),
        pltpu.VMEM((B, H), jnp.float32),
        pltpu.VMEM_SHARED((NP2, H), jnp.float32),
        pltpu.SemaphoreType.DMA,
    ],
)
def _sc_degree(dst_hbm, ones_hbm, zeros_hbm, zblk_hbm, out_hbm, dst_v, ones_v, acc, ssem):
    c = lax.axis_index("c")
    s = lax.axis_index("s")
    wid = c * NS + s
    pltpu.sync_copy(ones_hbm, ones_v)
    row0 = s * RPT
    pltpu.sync_copy(zeros_hbm, acc.at[pl.ds(row0, RPT)])
    pltpu.sync_copy(dst_hbm.at[wid], dst_v)
    plsc.subcore_barrier()

    def wait_s():
        pltpu.make_async_copy(zblk_hbm, ones_v, ssem).wait()

    def body(i, carry):
        @pl.when(i >= DW)
        def _():
            wait_s()

        pltpu.async_copy(ones_v, acc.at[dst_v.at[i]], ssem, add=True)
        return carry

    lax.fori_loop(0, EBP, body, 0)
    for _ in range(DW):
        wait_s()
    plsc.subcore_barrier()
    pltpu.sync_copy(acc.at[pl.ds(row0, RPT)], out_hbm.at[c, pl.ds(row0, RPT)])


@functools.partial(
    pl.kernel,
    out_type=jax.ShapeDtypeStruct((NC, NP2, H), jnp.float32),
    mesh=_mesh,
    scratch_types=[
        pltpu.VMEM((HEB, B), jnp.int32),
        pltpu.VMEM((HEB, B), jnp.int32),
        pltpu.VMEM((B, H), jnp.float32),
        pltpu.VMEM((B, H), jnp.float32),
        pltpu.VMEM_SHARED((NP2, H), jnp.float32),
        pltpu.SemaphoreType.DMA,
        pltpu.SemaphoreType.DMA,
    ],
)
def _sc_scatter(table_hbm, src_hbm, dst_hbm, zeros_hbm, zblk_hbm, out_hbm,
                src_v, dst_v, rows0, rows1, acc, gsem, ssem):
    c = lax.axis_index("c")
    s = lax.axis_index("s")
    wid = c * NS + s
    row0 = s * RPT
    pltpu.sync_copy(zeros_hbm, acc.at[pl.ds(row0, RPT)])

    def gather(i, buf):
        pltpu.async_copy(table_hbm.at[src_v.at[i]], buf, gsem)

    def wait_g(buf):
        pltpu.make_async_copy(zblk_hbm, buf, gsem).wait()

    def scat(i, buf):
        pltpu.sync_copy(buf, acc.at[dst_v.at[i]], add=True)

    for half in range(2):
        pltpu.sync_copy(src_hbm.at[wid, pl.ds(half * HEB, HEB)], src_v)
        pltpu.sync_copy(dst_hbm.at[wid, pl.ds(half * HEB, HEB)], dst_v)
        if half == 0:
            plsc.subcore_barrier()
        def blk(i, carry):
            pltpu.async_copy(table_hbm.at[src_v.at[i]], rows0, gsem).wait()
            scat(i, rows0)
            return carry

        lax.fori_loop(0, HEB, blk, 0)
    plsc.subcore_barrier()
    pltpu.sync_copy(acc.at[pl.ds(row0, RPT)], out_hbm.at[c, pl.ds(row0, RPT)])


BN = 1000
NBLK = N // BN


def _dinv(deg_ref):
    d = deg_ref[0, :, 0:1] + deg_ref[1, :, 0:1] + 1.0
    return lax.rsqrt(jnp.maximum(d, 1.0))


def _tc_prologue_body(deg_ref, x_ref, w_ref, out_ref):
    out_ref[...] = jnp.dot(x_ref[...], w_ref[...],
                           preferred_element_type=jnp.float32) * _dinv(deg_ref)


def _combine(deg_ref, s2_ref, hwp_ref, b_ref):
    dinv = _dinv(deg_ref)
    return (s2_ref[0] + s2_ref[1] + hwp_ref[...]) * dinv + b_ref[...], dinv


def _layernorm(t, g_ref, be_ref):
    mu = jnp.mean(t, axis=-1, keepdims=True)
    var = jnp.mean((t - mu) ** 2, axis=-1, keepdims=True)
    return (t - mu) * lax.rsqrt(var + 1e-5) * g_ref[...] + be_ref[...]


def _tc_layer0_body(deg_ref, s2_ref, hwp_ref, b_ref, w_ref, h_ref, hn_ref):
    t, dinv = _combine(deg_ref, s2_ref, hwp_ref, b_ref)
    h = jnp.maximum(t, 0.0)
    h_ref[...] = h
    hn_ref[...] = jnp.dot(h, w_ref[...], preferred_element_type=jnp.float32) * dinv


def _tc_mid_body(deg_ref, s2_ref, hwp_ref, hp_ref, g_ref, be_ref, b_ref, w_ref,
                 h_ref, hn_ref):
    t, dinv = _combine(deg_ref, s2_ref, hwp_ref, b_ref)
    h = jnp.maximum(_layernorm(t, g_ref, be_ref), 0.0) + hp_ref[...]
    h_ref[...] = h
    hn_ref[...] = jnp.dot(h, w_ref[...], preferred_element_type=jnp.float32) * dinv


def _tc_final_body(deg_ref, s2_ref, hwp_ref, hp_ref, g_ref, be_ref, b_ref,
                   batch_ref, fc1w_ref, fc1b_ref, fc2w_ref, fc2b_ref,
                   out_ref, acc, cnt):
    i = pl.program_id(0)
    t, _ = _combine(deg_ref, s2_ref, hwp_ref, b_ref)
    h = jnp.maximum(_layernorm(t, g_ref, be_ref), 0.0) + hp_ref[...]

    bt = batch_ref[0, 0, :]
    onehot = (bt[:, None] == lax.broadcasted_iota(jnp.int32, (1, G), 1)
              ).astype(jnp.float32)
    part = lax.dot_general(onehot, h, (((0,), (0,)), ((), ())),
                           preferred_element_type=jnp.float32)
    cpart = lax.dot_general(onehot, jnp.ones_like(h), (((0,), (0,)), ((), ())),
                            preferred_element_type=jnp.float32)

    @pl.when(i == 0)
    def _():
        acc[...] = jnp.zeros_like(acc)
        cnt[...] = jnp.zeros_like(cnt)

    acc[...] += part
    cnt[...] += cpart

    @pl.when(i == pl.num_programs(0) - 1)
    def _():
        pooled = acc[...] / jnp.maximum(cnt[...], 1.0)
        z = jnp.maximum(jnp.dot(pooled, fc1w_ref[...],
                                preferred_element_type=jnp.float32)
                        + fc1b_ref[...], 0.0)
        o = jnp.dot(z, fc2w_ref[...], preferred_element_type=jnp.float32) \
            + fc2b_ref[...]
        out_ref[...] = 1.0 / (1.0 + jnp.exp(-o))


_deg_spec = pl.BlockSpec((2, BN, H), lambda i: (0, i, 0))
_s2_spec = pl.BlockSpec((2, BN, H), lambda i: (0, i, 0))
_row_spec = pl.BlockSpec((BN, H), lambda i: (i, 0))
_vec_spec = pl.BlockSpec((1, H), lambda i: (0, 0))
_w_spec = pl.BlockSpec((H, H), lambda i: (0, 0))

_rowout = jax.ShapeDtypeStruct((N, H), jnp.float32)

_tc_prologue = pl.pallas_call(
    _tc_prologue_body, grid=(NBLK,),
    in_specs=[_deg_spec, _row_spec, _w_spec],
    out_specs=_row_spec, out_shape=_rowout)

_tc_layer0 = pl.pallas_call(
    _tc_layer0_body, grid=(NBLK,),
    in_specs=[_deg_spec, _s2_spec, _row_spec, _vec_spec, _w_spec],
    out_specs=(_row_spec, _row_spec), out_shape=(_rowout, _rowout))

_tc_mid = pl.pallas_call(
    _tc_mid_body, grid=(NBLK,),
    in_specs=[_deg_spec, _s2_spec, _row_spec, _row_spec, _vec_spec, _vec_spec,
              _vec_spec, _w_spec],
    out_specs=(_row_spec, _row_spec), out_shape=(_rowout, _rowout))

_tc_final = pl.pallas_call(
    _tc_final_body, grid=(NBLK,),
    in_specs=[_deg_spec, _s2_spec, _row_spec, _row_spec, _vec_spec, _vec_spec,
              _vec_spec,
              pl.BlockSpec((1, 1, BN), lambda i: (i, 0, 0)),
              pl.BlockSpec((H, 64), lambda i: (0, 0)),
              pl.BlockSpec((1, 64), lambda i: (0, 0)),
              pl.BlockSpec((64, H), lambda i: (0, 0)),
              _vec_spec],
    out_specs=pl.BlockSpec((G, H), lambda i: (0, 0)),
    out_shape=jax.ShapeDtypeStruct((G, H), jnp.float32),
    scratch_shapes=[pltpu.VMEM((G, H), jnp.float32),
                    pltpu.VMEM((G, H), jnp.float32)])


def kernel(x, edge_index, batch, Ws, bs, gammas, betas, fc1_W, fc1_b, fc2_W,
           fc2_b):
    pad = EPWP - EPW
    src = jnp.pad(edge_index[0].astype(jnp.int32).reshape(NW, EPW),
                  ((0, 0), (0, pad))).reshape(NW, EBP, B)
    padrows = jnp.broadcast_to(N + jnp.arange(pad, dtype=jnp.int32) % (NP2 - N),
                               (NW, pad))
    dst = jnp.concatenate(
        [edge_index[1].astype(jnp.int32).reshape(NW, EPW), padrows],
        axis=1).reshape(NW, EBP, B)
    batch3 = batch.astype(jnp.int32).reshape(NBLK, 1, BN)

    ones16 = jnp.ones((B, H), jnp.float32)
    zerosH = jnp.zeros((RPT, H), jnp.float32)
    zblk = jnp.zeros((B, H), jnp.float32)

    bs2 = bs.reshape(8, 1, H)
    g2 = gammas.reshape(7, 1, H)
    be2 = betas.reshape(7, 1, H)
    fc1b2 = fc1_b.reshape(1, 64)
    fc2wp = jnp.pad(fc2_W, ((0, 0), (0, H - 1)))
    fc2bp = jnp.pad(fc2_b, (0, H - 1)).reshape(1, H)

    deg2 = _sc_degree(dst, ones16, zerosH, zblk)
    hwp = _tc_prologue(deg2, x, Ws[0])

    h = None
    out = None
    for i in range(8):
        s2 = _sc_scatter(hwp, src, dst, zerosH, zblk)
        if i == 0:
            h, hwp = _tc_layer0(deg2, s2, hwp, bs2[0], Ws[1])
        elif i < 7:
            h, hwp = _tc_mid(deg2, s2, hwp, h, g2[i - 1], be2[i - 1], bs2[i],
                             Ws[i + 1])
        else:
            out = _tc_final(deg2, s2, hwp, h, g2[6], be2[6], bs2[7], batch3,
                            fc1_W, fc1b2, fc2wp, fc2bp)
    return out[:, :1]


# speedup vs baseline: 2.2611x; 2.2609x over previous
import functools

import jax
import jax.numpy as jnp
from jax import lax
from jax.experimental import pallas as pl
from jax.experimental.pallas import tpu as pltpu
from jax.experimental.pallas import tpu_sc as plsc

N = 10000
E = 320000
H = 128
G = 16

NC = 2
NS = 16
NW = NC * NS
B = 80
EPW = E // NW
EB = EPW // B
NP2 = 10240
RPT = NP2 // NS
ZR = 128

_mesh = plsc.VectorSubcoreMesh(core_axis_name="c", subcore_axis_name="s")


@functools.partial(
    pl.kernel,
    out_type=jax.ShapeDtypeStruct((NC, NP2, H), jnp.float32),
    mesh=_mesh,
    scratch_types=[
        pltpu.VMEM((EB, B), jnp.int32),
        pltpu.VMEM((B, H), jnp.float32),
        pltpu.VMEM_SHARED((NP2, H), jnp.float32),
    ],
)
def _sc_degree(dst_hbm, ones_hbm, zeros_hbm, out_hbm, dst_v, ones_v, acc):
    c = lax.axis_index("c")
    s = lax.axis_index("s")
    wid = c * NS + s
    pltpu.sync_copy(ones_hbm, ones_v)
    row0 = s * RPT
    pltpu.sync_copy(zeros_hbm, acc.at[pl.ds(row0, RPT)])
    pltpu.sync_copy(dst_hbm.at[wid], dst_v)
    plsc.subcore_barrier()

    def body(i, carry):
        pltpu.sync_copy(ones_v, acc.at[dst_v.at[i]], add=True)
        return carry

    lax.fori_loop(0, EB, body, 0)
    plsc.subcore_barrier()
    for k in range(RPT // ZR):
        r = row0 + k * ZR
        pltpu.sync_copy(acc.at[pl.ds(r, ZR)], out_hbm.at[c, pl.ds(r, ZR)])


@functools.partial(
    pl.kernel,
    out_type=jax.ShapeDtypeStruct((NC, NP2, H), jnp.float32),
    mesh=_mesh,
    scratch_types=[
        pltpu.VMEM((EB, B), jnp.int32),
        pltpu.VMEM((EB, B), jnp.int32),
        pltpu.VMEM((B, H), jnp.float32),
        pltpu.VMEM_SHARED((NP2, H), jnp.float32),
        pltpu.SemaphoreType.DMA,
    ],
)
def _sc_scatter(table_hbm, src_hbm, dst_hbm, zeros_hbm, out_hbm,
                src_v, dst_v, rows_v, acc, sem):
    c = lax.axis_index("c")
    s = lax.axis_index("s")
    wid = c * NS + s
    row0 = s * RPT
    pltpu.sync_copy(zeros_hbm, acc.at[pl.ds(row0, RPT)])
    pltpu.sync_copy(src_hbm.at[wid], src_v)
    pltpu.sync_copy(dst_hbm.at[wid], dst_v)
    plsc.subcore_barrier()

    def body(i, carry):
        pltpu.async_copy(table_hbm.at[src_v.at[i]], rows_v, sem).wait()
        pltpu.sync_copy(rows_v, acc.at[dst_v.at[i]], add=True)
        return carry

    lax.fori_loop(0, EB, body, 0)
    plsc.subcore_barrier()
    for k in range(RPT // ZR):
        r = row0 + k * ZR
        pltpu.sync_copy(acc.at[pl.ds(r, ZR)], out_hbm.at[c, pl.ds(r, ZR)])


BN = 1000
NBLK = N // BN


def _dinv(deg_ref):
    d = deg_ref[0, :, 0:1] + deg_ref[1, :, 0:1] + 1.0
    return lax.rsqrt(jnp.maximum(d, 1.0))


def _tc_prologue_body(deg_ref, x_ref, w_ref, out_ref):
    out_ref[...] = jnp.dot(x_ref[...], w_ref[...],
                           preferred_element_type=jnp.float32) * _dinv(deg_ref)


def _combine(deg_ref, s2_ref, hwp_ref, b_ref):
    dinv = _dinv(deg_ref)
    return (s2_ref[0] + s2_ref[1] + hwp_ref[...]) * dinv + b_ref[...], dinv


def _layernorm(t, g_ref, be_ref):
    mu = jnp.mean(t, axis=-1, keepdims=True)
    var = jnp.mean((t - mu) ** 2, axis=-1, keepdims=True)
    return (t - mu) * lax.rsqrt(var + 1e-5) * g_ref[...] + be_ref[...]


def _tc_layer0_body(deg_ref, s2_ref, hwp_ref, b_ref, w_ref, h_ref, hn_ref):
    t, dinv = _combine(deg_ref, s2_ref, hwp_ref, b_ref)
    h = jnp.maximum(t, 0.0)
    h_ref[...] = h
    hn_ref[...] = jnp.dot(h, w_ref[...], preferred_element_type=jnp.float32) * dinv


def _tc_mid_body(deg_ref, s2_ref, hwp_ref, hp_ref, g_ref, be_ref, b_ref, w_ref,
                 h_ref, hn_ref):
    t, dinv = _combine(deg_ref, s2_ref, hwp_ref, b_ref)
    h = jnp.maximum(_layernorm(t, g_ref, be_ref), 0.0) + hp_ref[...]
    h_ref[...] = h
    hn_ref[...] = jnp.dot(h, w_ref[...], preferred_element_type=jnp.float32) * dinv


def _tc_final_body(deg_ref, s2_ref, hwp_ref, hp_ref, g_ref, be_ref, b_ref,
                   batch_ref, fc1w_ref, fc1b_ref, fc2w_ref, fc2b_ref,
                   out_ref, acc, cnt):
    i = pl.program_id(0)
    t, _ = _combine(deg_ref, s2_ref, hwp_ref, b_ref)
    h = jnp.maximum(_layernorm(t, g_ref, be_ref), 0.0) + hp_ref[...]

    bt = batch_ref[0, 0, :]
    onehot = (bt[:, None] == lax.broadcasted_iota(jnp.int32, (1, G), 1)
              ).astype(jnp.float32)
    part = lax.dot_general(onehot, h, (((0,), (0,)), ((), ())),
                           preferred_element_type=jnp.float32)
    cpart = lax.dot_general(onehot, jnp.ones_like(h), (((0,), (0,)), ((), ())),
                            preferred_element_type=jnp.float32)

    @pl.when(i == 0)
    def _():
        acc[...] = jnp.zeros_like(acc)
        cnt[...] = jnp.zeros_like(cnt)

    acc[...] += part
    cnt[...] += cpart

    @pl.when(i == pl.num_programs(0) - 1)
    def _():
        pooled = acc[...] / jnp.maximum(cnt[...], 1.0)
        z = jnp.maximum(jnp.dot(pooled, fc1w_ref[...],
                                preferred_element_type=jnp.float32)
                        + fc1b_ref[...], 0.0)
        o = jnp.dot(z, fc2w_ref[...], preferred_element_type=jnp.float32) \
            + fc2b_ref[...]
        out_ref[...] = 1.0 / (1.0 + jnp.exp(-o))


_deg_spec = pl.BlockSpec((2, BN, H), lambda i: (0, i, 0))
_s2_spec = pl.BlockSpec((2, BN, H), lambda i: (0, i, 0))
_row_spec = pl.BlockSpec((BN, H), lambda i: (i, 0))
_vec_spec = pl.BlockSpec((1, H), lambda i: (0, 0))
_w_spec = pl.BlockSpec((H, H), lambda i: (0, 0))

_rowout = jax.ShapeDtypeStruct((N, H), jnp.float32)

_tc_prologue = pl.pallas_call(
    _tc_prologue_body, grid=(NBLK,),
    in_specs=[_deg_spec, _row_spec, _w_spec],
    out_specs=_row_spec, out_shape=_rowout)

_tc_layer0 = pl.pallas_call(
    _tc_layer0_body, grid=(NBLK,),
    in_specs=[_deg_spec, _s2_spec, _row_spec, _vec_spec, _w_spec],
    out_specs=(_row_spec, _row_spec), out_shape=(_rowout, _rowout))

_tc_mid = pl.pallas_call(
    _tc_mid_body, grid=(NBLK,),
    in_specs=[_deg_spec, _s2_spec, _row_spec, _row_spec, _vec_spec, _vec_spec,
              _vec_spec, _w_spec],
    out_specs=(_row_spec, _row_spec), out_shape=(_rowout, _rowout))

_tc_final = pl.pallas_call(
    _tc_final_body, grid=(NBLK,),
    in_specs=[_deg_spec, _s2_spec, _row_spec, _row_spec, _vec_spec, _vec_spec,
              _vec_spec,
              pl.BlockSpec((1, 1, BN), lambda i: (i, 0, 0)),
              pl.BlockSpec((H, 64), lambda i: (0, 0)),
              pl.BlockSpec((1, 64), lambda i: (0, 0)),
              pl.BlockSpec((64, H), lambda i: (0, 0)),
              _vec_spec],
    out_specs=pl.BlockSpec((G, H), lambda i: (0, 0)),
    out_shape=jax.ShapeDtypeStruct((G, H), jnp.float32),
    scratch_shapes=[pltpu.VMEM((G, H), jnp.float32),
                    pltpu.VMEM((G, H), jnp.float32)])


def kernel(x, edge_index, batch, Ws, bs, gammas, betas, fc1_W, fc1_b, fc2_W,
           fc2_b):
    src = edge_index[0].astype(jnp.int32).reshape(NW, EB, B)
    dst = edge_index[1].astype(jnp.int32).reshape(NW, EB, B)
    batch3 = batch.astype(jnp.int32).reshape(NBLK, 1, BN)

    ones16 = jnp.ones((B, H), jnp.float32)
    zerosH = jnp.zeros((RPT, H), jnp.float32)

    bs2 = bs.reshape(8, 1, H)
    g2 = gammas.reshape(7, 1, H)
    be2 = betas.reshape(7, 1, H)
    fc1b2 = fc1_b.reshape(1, 64)
    fc2wp = jnp.pad(fc2_W, ((0, 0), (0, H - 1)))
    fc2bp = jnp.pad(fc2_b, (0, H - 1)).reshape(1, H)

    deg2 = _sc_degree(dst, ones16, zerosH)
    hwp = _tc_prologue(deg2, x, Ws[0])

    h = None
    out = None
    for i in range(8):
        s2 = _sc_scatter(hwp, src, dst, zerosH)
        if i == 0:
            h, hwp = _tc_layer0(deg2, s2, hwp, bs2[0], Ws[1])
        elif i < 7:
            h, hwp = _tc_mid(deg2, s2, hwp, h, g2[i - 1], be2[i - 1], bs2[i],
                             Ws[i + 1])
        else:
            out = _tc_final(deg2, s2, hwp, h, g2[6], be2[6], bs2[7], batch3,
                            fc1_W, fc1b2, fc2wp, fc2bp)
    return out[:, :1]


# speedup vs baseline: 2.4274x; 1.0735x over previous
import functools

import jax
import jax.numpy as jnp
from jax import lax
from jax.experimental import pallas as pl
from jax.experimental.pallas import tpu as pltpu
from jax.experimental.pallas import tpu_sc as plsc

N = 10000
E = 320000
H = 128
G = 16

NC = 2
NS = 16
NW = NC * NS
B = 80
EPW = E // NW
EB = EPW // B
B2 = 100
HEB = 50
NP2 = 10240
RPT = NP2 // NS
ZR = 128

_mesh = plsc.VectorSubcoreMesh(core_axis_name="c", subcore_axis_name="s")


@functools.partial(
    pl.kernel,
    out_type=jax.ShapeDtypeStruct((NC, NP2, H), jnp.float32),
    mesh=_mesh,
    scratch_types=[
        pltpu.VMEM((EB, B), jnp.int32),
        pltpu.VMEM((B, H), jnp.float32),
        pltpu.VMEM_SHARED((NP2, H), jnp.float32),
    ],
)
def _sc_degree(dst_hbm, ones_hbm, zeros_hbm, out_hbm, dst_v, ones_v, acc):
    c = lax.axis_index("c")
    s = lax.axis_index("s")
    wid = c * NS + s
    pltpu.sync_copy(ones_hbm, ones_v)
    row0 = s * RPT
    pltpu.sync_copy(zeros_hbm, acc.at[pl.ds(row0, RPT)])
    pltpu.sync_copy(dst_hbm.at[wid], dst_v)
    plsc.subcore_barrier()

    def body(i, carry):
        pltpu.sync_copy(ones_v, acc.at[dst_v.at[i]], add=True)
        return carry

    lax.fori_loop(0, EB, body, 0)
    plsc.subcore_barrier()
    for k in range(RPT // ZR):
        r = row0 + k * ZR
        pltpu.sync_copy(acc.at[pl.ds(r, ZR)], out_hbm.at[c, pl.ds(r, ZR)])


@functools.partial(
    pl.kernel,
    out_type=jax.ShapeDtypeStruct((NC, NP2, H), jnp.float32),
    mesh=_mesh,
    scratch_types=[
        pltpu.VMEM((HEB, B2), jnp.int32),
        pltpu.VMEM((HEB, B2), jnp.int32),
        pltpu.VMEM((B2, H), jnp.float32),
        pltpu.VMEM_SHARED((NP2, H), jnp.float32),
        pltpu.SemaphoreType.DMA,
    ],
)
def _sc_scatter(table_hbm, src_hbm, dst_hbm, zeros_hbm, out_hbm,
                src_v, dst_v, rows_v, acc, sem):
    c = lax.axis_index("c")
    s = lax.axis_index("s")
    wid = c * NS + s
    row0 = s * RPT
    pltpu.sync_copy(zeros_hbm, acc.at[pl.ds(row0, RPT)])

    def body(i, carry):
        pltpu.async_copy(table_hbm.at[src_v.at[i]], rows_v, sem).wait()
        pltpu.sync_copy(rows_v, acc.at[dst_v.at[i]], add=True)
        return carry

    for half in range(2):
        pltpu.sync_copy(src_hbm.at[wid, half], src_v)
        pltpu.sync_copy(dst_hbm.at[wid, half], dst_v)
        if half == 0:
            plsc.subcore_barrier()
        lax.fori_loop(0, HEB, body, 0)
    plsc.subcore_barrier()
    for k in range(RPT // ZR):
        r = row0 + k * ZR
        pltpu.sync_copy(acc.at[pl.ds(r, ZR)], out_hbm.at[c, pl.ds(r, ZR)])


BN = 1000
NBLK = N // BN


def _dinv(deg_ref):
    d = deg_ref[0, :, 0:1] + deg_ref[1, :, 0:1] + 1.0
    return lax.rsqrt(jnp.maximum(d, 1.0))


def _tc_prologue_body(deg_ref, x_ref, w_ref, out_ref):
    out_ref[...] = jnp.dot(x_ref[...], w_ref[...],
                           preferred_element_type=jnp.float32) * _dinv(deg_ref)


def _combine(deg_ref, s2_ref, hwp_ref, b_ref):
    dinv = _dinv(deg_ref)
    return (s2_ref[0] + s2_ref[1] + hwp_ref[...]) * dinv + b_ref[...], dinv


def _layernorm(t, g_ref, be_ref):
    mu = jnp.mean(t, axis=-1, keepdims=True)
    var = jnp.mean((t - mu) ** 2, axis=-1, keepdims=True)
    return (t - mu) * lax.rsqrt(var + 1e-5) * g_ref[...] + be_ref[...]


def _tc_layer0_body(deg_ref, s2_ref, hwp_ref, b_ref, w_ref, h_ref, hn_ref):
    t, dinv = _combine(deg_ref, s2_ref, hwp_ref, b_ref)
    h = jnp.maximum(t, 0.0)
    h_ref[...] = h
    hn_ref[...] = jnp.dot(h, w_ref[...], preferred_element_type=jnp.float32) * dinv


def _tc_mid_body(deg_ref, s2_ref, hwp_ref, hp_ref, g_ref, be_ref, b_ref, w_ref,
                 h_ref, hn_ref):
    t, dinv = _combine(deg_ref, s2_ref, hwp_ref, b_ref)
    h = jnp.maximum(_layernorm(t, g_ref, be_ref), 0.0) + hp_ref[...]
    h_ref[...] = h
    hn_ref[...] = jnp.dot(h, w_ref[...], preferred_element_type=jnp.float32) * dinv


def _tc_final_body(deg_ref, s2_ref, hwp_ref, hp_ref, g_ref, be_ref, b_ref,
                   batch_ref, fc1w_ref, fc1b_ref, fc2w_ref, fc2b_ref,
                   out_ref, acc, cnt):
    i = pl.program_id(0)
    t, _ = _combine(deg_ref, s2_ref, hwp_ref, b_ref)
    h = jnp.maximum(_layernorm(t, g_ref, be_ref), 0.0) + hp_ref[...]

    bt = batch_ref[0, 0, :]
    onehot = (bt[:, None] == lax.broadcasted_iota(jnp.int32, (1, G), 1)
              ).astype(jnp.float32)
    part = lax.dot_general(onehot, h, (((0,), (0,)), ((), ())),
                           preferred_element_type=jnp.float32)
    cpart = lax.dot_general(onehot, jnp.ones_like(h), (((0,), (0,)), ((), ())),
                            preferred_element_type=jnp.float32)

    @pl.when(i == 0)
    def _():
        acc[...] = jnp.zeros_like(acc)
        cnt[...] = jnp.zeros_like(cnt)

    acc[...] += part
    cnt[...] += cpart

    @pl.when(i == pl.num_programs(0) - 1)
    def _():
        pooled = acc[...] / jnp.maximum(cnt[...], 1.0)
        z = jnp.maximum(jnp.dot(pooled, fc1w_ref[...],
                                preferred_element_type=jnp.float32)
                        + fc1b_ref[...], 0.0)
        o = jnp.dot(z, fc2w_ref[...], preferred_element_type=jnp.float32) \
            + fc2b_ref[...]
        out_ref[...] = 1.0 / (1.0 + jnp.exp(-o))


_deg_spec = pl.BlockSpec((2, BN, H), lambda i: (0, i, 0))
_s2_spec = pl.BlockSpec((2, BN, H), lambda i: (0, i, 0))
_row_spec = pl.BlockSpec((BN, H), lambda i: (i, 0))
_vec_spec = pl.BlockSpec((1, H), lambda i: (0, 0))
_w_spec = pl.BlockSpec((H, H), lambda i: (0, 0))

_rowout = jax.ShapeDtypeStruct((N, H), jnp.float32)

_tc_prologue = pl.pallas_call(
    _tc_prologue_body, grid=(NBLK,),
    in_specs=[_deg_spec, _row_spec, _w_spec],
    out_specs=_row_spec, out_shape=_rowout)

_tc_layer0 = pl.pallas_call(
    _tc_layer0_body, grid=(NBLK,),
    in_specs=[_deg_spec, _s2_spec, _row_spec, _vec_spec, _w_spec],
    out_specs=(_row_spec, _row_spec), out_shape=(_rowout, _rowout))

_tc_mid = pl.pallas_call(
    _tc_mid_body, grid=(NBLK,),
    in_specs=[_deg_spec, _s2_spec, _row_spec, _row_spec, _vec_spec, _vec_spec,
              _vec_spec, _w_spec],
    out_specs=(_row_spec, _row_spec), out_shape=(_rowout, _rowout))

_tc_final = pl.pallas_call(
    _tc_final_body, grid=(NBLK,),
    in_specs=[_deg_spec, _s2_spec, _row_spec, _row_spec, _vec_spec, _vec_spec,
              _vec_spec,
              pl.BlockSpec((1, 1, BN), lambda i: (i, 0, 0)),
              pl.BlockSpec((H, 64), lambda i: (0, 0)),
              pl.BlockSpec((1, 64), lambda i: (0, 0)),
              pl.BlockSpec((64, H), lambda i: (0, 0)),
              _vec_spec],
    out_specs=pl.BlockSpec((G, H), lambda i: (0, 0)),
    out_shape=jax.ShapeDtypeStruct((G, H), jnp.float32),
    scratch_shapes=[pltpu.VMEM((G, H), jnp.float32),
                    pltpu.VMEM((G, H), jnp.float32)])


def kernel(x, edge_index, batch, Ws, bs, gammas, betas, fc1_W, fc1_b, fc2_W,
           fc2_b):
    src = edge_index[0].astype(jnp.int32).reshape(NW, 2, HEB, B2)
    dst = edge_index[1].astype(jnp.int32).reshape(NW, 2, HEB, B2)
    dstd = edge_index[1].astype(jnp.int32).reshape(NW, EB, B)
    batch3 = batch.astype(jnp.int32).reshape(NBLK, 1, BN)

    ones16 = jnp.ones((B, H), jnp.float32)
    zerosH = jnp.zeros((RPT, H), jnp.float32)

    bs2 = bs.reshape(8, 1, H)
    g2 = gammas.reshape(7, 1, H)
    be2 = betas.reshape(7, 1, H)
    fc1b2 = fc1_b.reshape(1, 64)
    fc2wp = jnp.pad(fc2_W, ((0, 0), (0, H - 1)))
    fc2bp = jnp.pad(fc2_b, (0, H - 1)).reshape(1, H)

    deg2 = _sc_degree(dstd, ones16, zerosH)
    hwp = _tc_prologue(deg2, x, Ws[0])

    h = None
    out = None
    for i in range(8):
        s2 = _sc_scatter(hwp, src, dst, zerosH)
        if i == 0:
            h, hwp = _tc_layer0(deg2, s2, hwp, bs2[0], Ws[1])
        elif i < 7:
            h, hwp = _tc_mid(deg2, s2, hwp, h, g2[i - 1], be2[i - 1], bs2[i],
                             Ws[i + 1])
        else:
            out = _tc_final(deg2, s2, hwp, h, g2[6], be2[6], bs2[7], batch3,
                            fc1_W, fc1b2, fc2wp, fc2bp)
    return out[:, :1]


# speedup vs baseline: 3.0850x; 1.2709x over previous
import functools

import jax
import jax.numpy as jnp
from jax import lax
from jax.experimental import pallas as pl
from jax.experimental.pallas import tpu as pltpu
from jax.experimental.pallas import tpu_sc as plsc

N = 10000
E = 320000
H = 128
G = 16

NC = 2
NS = 16
NW = NC * NS
B = 80
EPW = E // NW
EB = EPW // B
B2 = 100
HEB = 50
NP2 = 10240
RPT = NP2 // NS
ZR = 128

_mesh = plsc.VectorSubcoreMesh(core_axis_name="c", subcore_axis_name="s")


@functools.partial(
    pl.kernel,
    out_type=jax.ShapeDtypeStruct((NC, NP2, H), jnp.float32),
    mesh=_mesh,
    scratch_types=[
        pltpu.VMEM((EB, B), jnp.int32),
        pltpu.VMEM((B, H), jnp.float32),
        pltpu.VMEM_SHARED((NP2, H), jnp.float32),
    ],
)
def _sc_degree(dst_hbm, ones_hbm, zeros_hbm, out_hbm, dst_v, ones_v, acc):
    c = lax.axis_index("c")
    s = lax.axis_index("s")
    wid = c * NS + s
    pltpu.sync_copy(ones_hbm, ones_v)
    row0 = s * RPT
    pltpu.sync_copy(zeros_hbm, acc.at[pl.ds(row0, RPT)])
    pltpu.sync_copy(dst_hbm.at[wid], dst_v)
    plsc.subcore_barrier()

    def body(i, carry):
        pltpu.sync_copy(ones_v, acc.at[dst_v.at[i]], add=True)
        return carry

    lax.fori_loop(0, EB, body, 0)
    plsc.subcore_barrier()
    for k in range(RPT // ZR):
        r = row0 + k * ZR
        pltpu.sync_copy(acc.at[pl.ds(r, ZR)], out_hbm.at[c, pl.ds(r, ZR)])


@functools.partial(
    pl.kernel,
    out_type=jax.ShapeDtypeStruct((NC, NP2, H), jnp.float32),
    mesh=_mesh,
    scratch_types=[
        pltpu.VMEM((HEB, B2), jnp.int32),
        pltpu.VMEM((HEB, B2), jnp.int32),
        pltpu.VMEM((B2, H), jnp.float32),
        pltpu.VMEM((B2, H), jnp.float32),
        pltpu.VMEM_SHARED((NP2, H), jnp.float32),
        pltpu.SemaphoreType.DMA,
    ],
)
def _sc_scatter(table_hbm, src_hbm, dst_hbm, zeros_hbm, zblk_hbm, out_hbm,
                src_v, dst_v, rows0, rows1, acc, gsem):
    c = lax.axis_index("c")
    s = lax.axis_index("s")
    wid = c * NS + s
    row0 = s * RPT
    pltpu.sync_copy(zeros_hbm, acc.at[pl.ds(row0, RPT)])

    def gather(i, buf):
        pltpu.async_copy(table_hbm.at[src_v.at[i]], buf, gsem)

    def wait_g(buf):
        pltpu.make_async_copy(zblk_hbm, buf, gsem).wait()

    def scat(i, buf):
        pltpu.sync_copy(buf, acc.at[dst_v.at[i]], add=True)

    NG2 = HEB // 2
    for half in range(2):
        pltpu.sync_copy(src_hbm.at[wid, half], src_v)
        pltpu.sync_copy(dst_hbm.at[wid, half], dst_v)
        if half == 0:
            plsc.subcore_barrier()
        gather(0, rows0)

        def grp(g, carry):
            i0 = 2 * g
            wait_g(rows0)
            gather(i0 + 1, rows1)
            scat(i0, rows0)

            wait_g(rows1)

            @pl.when(g < NG2 - 1)
            def _():
                gather(i0 + 2, rows0)

            scat(i0 + 1, rows1)
            return carry

        lax.fori_loop(0, NG2, grp, 0)
    plsc.subcore_barrier()
    for k in range(RPT // ZR):
        r = row0 + k * ZR
        pltpu.sync_copy(acc.at[pl.ds(r, ZR)], out_hbm.at[c, pl.ds(r, ZR)])


BN = 1000
NBLK = N // BN


def _dinv(deg_ref):
    d = deg_ref[0, :, 0:1] + deg_ref[1, :, 0:1] + 1.0
    return lax.rsqrt(jnp.maximum(d, 1.0))


def _tc_prologue_body(deg_ref, x_ref, w_ref, out_ref):
    out_ref[...] = jnp.dot(x_ref[...], w_ref[...],
                           preferred_element_type=jnp.float32) * _dinv(deg_ref)


def _combine(deg_ref, s2_ref, hwp_ref, b_ref):
    dinv = _dinv(deg_ref)
    return (s2_ref[0] + s2_ref[1] + hwp_ref[...]) * dinv + b_ref[...], dinv


def _layernorm(t, g_ref, be_ref):
    mu = jnp.mean(t, axis=-1, keepdims=True)
    var = jnp.mean((t - mu) ** 2, axis=-1, keepdims=True)
    return (t - mu) * lax.rsqrt(var + 1e-5) * g_ref[...] + be_ref[...]


def _tc_layer0_body(deg_ref, s2_ref, hwp_ref, b_ref, w_ref, h_ref, hn_ref):
    t, dinv = _combine(deg_ref, s2_ref, hwp_ref, b_ref)
    h = jnp.maximum(t, 0.0)
    h_ref[...] = h
    hn_ref[...] = jnp.dot(h, w_ref[...], preferred_element_type=jnp.float32) * dinv


def _tc_mid_body(deg_ref, s2_ref, hwp_ref, hp_ref, g_ref, be_ref, b_ref, w_ref,
                 h_ref, hn_ref):
    t, dinv = _combine(deg_ref, s2_ref, hwp_ref, b_ref)
    h = jnp.maximum(_layernorm(t, g_ref, be_ref), 0.0) + hp_ref[...]
    h_ref[...] = h
    hn_ref[...] = jnp.dot(h, w_ref[...], preferred_element_type=jnp.float32) * dinv


def _tc_final_body(deg_ref, s2_ref, hwp_ref, hp_ref, g_ref, be_ref, b_ref,
                   batch_ref, fc1w_ref, fc1b_ref, fc2w_ref, fc2b_ref,
                   out_ref, acc, cnt):
    i = pl.program_id(0)
    t, _ = _combine(deg_ref, s2_ref, hwp_ref, b_ref)
    h = jnp.maximum(_layernorm(t, g_ref, be_ref), 0.0) + hp_ref[...]

    bt = batch_ref[0, 0, :]
    onehot = (bt[:, None] == lax.broadcasted_iota(jnp.int32, (1, G), 1)
              ).astype(jnp.float32)
    part = lax.dot_general(onehot, h, (((0,), (0,)), ((), ())),
                           preferred_element_type=jnp.float32)
    cpart = lax.dot_general(onehot, jnp.ones_like(h), (((0,), (0,)), ((), ())),
                            preferred_element_type=jnp.float32)

    @pl.when(i == 0)
    def _():
        acc[...] = jnp.zeros_like(acc)
        cnt[...] = jnp.zeros_like(cnt)

    acc[...] += part
    cnt[...] += cpart

    @pl.when(i == pl.num_programs(0) - 1)
    def _():
        pooled = acc[...] / jnp.maximum(cnt[...], 1.0)
        z = jnp.maximum(jnp.dot(pooled, fc1w_ref[...],
                                preferred_element_type=jnp.float32)
                        + fc1b_ref[...], 0.0)
        o = jnp.dot(z, fc2w_ref[...], preferred_element_type=jnp.float32) \
            + fc2b_ref[...]
        out_ref[...] = 1.0 / (1.0 + jnp.exp(-o))


_deg_spec = pl.BlockSpec((2, BN, H), lambda i: (0, i, 0))
_s2_spec = pl.BlockSpec((2, BN, H), lambda i: (0, i, 0))
_row_spec = pl.BlockSpec((BN, H), lambda i: (i, 0))
_vec_spec = pl.BlockSpec((1, H), lambda i: (0, 0))
_w_spec = pl.BlockSpec((H, H), lambda i: (0, 0))

_rowout = jax.ShapeDtypeStruct((N, H), jnp.float32)

_tc_prologue = pl.pallas_call(
    _tc_prologue_body, grid=(NBLK,),
    in_specs=[_deg_spec, _row_spec, _w_spec],
    out_specs=_row_spec, out_shape=_rowout)

_tc_layer0 = pl.pallas_call(
    _tc_layer0_body, grid=(NBLK,),
    in_specs=[_deg_spec, _s2_spec, _row_spec, _vec_spec, _w_spec],
    out_specs=(_row_spec, _row_spec), out_shape=(_rowout, _rowout))

_tc_mid = pl.pallas_call(
    _tc_mid_body, grid=(NBLK,),
    in_specs=[_deg_spec, _s2_spec, _row_spec, _row_spec, _vec_spec, _vec_spec,
              _vec_spec, _w_spec],
    out_specs=(_row_spec, _row_spec), out_shape=(_rowout, _rowout))

_tc_final = pl.pallas_call(
    _tc_final_body, grid=(NBLK,),
    in_specs=[_deg_spec, _s2_spec, _row_spec, _row_spec, _vec_spec, _vec_spec,
              _vec_spec,
              pl.BlockSpec((1, 1, BN), lambda i: (i, 0, 0)),
              pl.BlockSpec((H, 64), lambda i: (0, 0)),
              pl.BlockSpec((1, 64), lambda i: (0, 0)),
              pl.BlockSpec((64, H), lambda i: (0, 0)),
              _vec_spec],
    out_specs=pl.BlockSpec((G, H), lambda i: (0, 0)),
    out_shape=jax.ShapeDtypeStruct((G, H), jnp.float32),
    scratch_shapes=[pltpu.VMEM((G, H), jnp.float32),
                    pltpu.VMEM((G, H), jnp.float32)])


def kernel(x, edge_index, batch, Ws, bs, gammas, betas, fc1_W, fc1_b, fc2_W,
           fc2_b):
    src = edge_index[0].astype(jnp.int32).reshape(NW, 2, HEB, B2)
    dst = edge_index[1].astype(jnp.int32).reshape(NW, 2, HEB, B2)
    dstd = edge_index[1].astype(jnp.int32).reshape(NW, EB, B)
    batch3 = batch.astype(jnp.int32).reshape(NBLK, 1, BN)

    ones16 = jnp.ones((B, H), jnp.float32)
    zerosH = jnp.zeros((RPT, H), jnp.float32)
    zblk = jnp.zeros((B2, H), jnp.float32)

    bs2 = bs.reshape(8, 1, H)
    g2 = gammas.reshape(7, 1, H)
    be2 = betas.reshape(7, 1, H)
    fc1b2 = fc1_b.reshape(1, 64)
    fc2wp = jnp.pad(fc2_W, ((0, 0), (0, H - 1)))
    fc2bp = jnp.pad(fc2_b, (0, H - 1)).reshape(1, H)

    deg2 = _sc_degree(dstd, ones16, zerosH)
    hwp = _tc_prologue(deg2, x, Ws[0])

    h = None
    out = None
    for i in range(8):
        s2 = _sc_scatter(hwp, src, dst, zerosH, zblk)
        if i == 0:
            h, hwp = _tc_layer0(deg2, s2, hwp, bs2[0], Ws[1])
        elif i < 7:
            h, hwp = _tc_mid(deg2, s2, hwp, h, g2[i - 1], be2[i - 1], bs2[i],
                             Ws[i + 1])
        else:
            out = _tc_final(deg2, s2, hwp, h, g2[6], be2[6], bs2[7], batch3,
                            fc1_W, fc1b2, fc2wp, fc2bp)
    return out[:, :1]


# speedup vs baseline: 3.0999x; 1.0048x over previous
import functools

import jax
import jax.numpy as jnp
from jax import lax
from jax.experimental import pallas as pl
from jax.experimental.pallas import tpu as pltpu
from jax.experimental.pallas import tpu_sc as plsc

N = 10000
E = 320000
H = 128
G = 16

NC = 2
NS = 16
NW = NC * NS
B = 80
EPW = E // NW
EB = EPW // B
B2 = 100
HEB = 50
DW = 4
NP2 = 10240
RPT = NP2 // NS
ZR = 128

_mesh = plsc.VectorSubcoreMesh(core_axis_name="c", subcore_axis_name="s")


@functools.partial(
    pl.kernel,
    out_type=jax.ShapeDtypeStruct((NC, NP2, H), jnp.float32),
    mesh=_mesh,
    scratch_types=[
        pltpu.VMEM((HEB, B2), jnp.int32),
        pltpu.VMEM((B2, H), jnp.float32),
        pltpu.VMEM_SHARED((NP2, H), jnp.float32),
        pltpu.SemaphoreType.DMA,
    ],
)
def _sc_degree(dst_hbm, ones_hbm, zeros_hbm, zblk_hbm, out_hbm,
               dst_v, ones_v, acc, ssem):
    c = lax.axis_index("c")
    s = lax.axis_index("s")
    wid = c * NS + s
    pltpu.sync_copy(ones_hbm, ones_v)
    row0 = s * RPT
    pltpu.sync_copy(zeros_hbm, acc.at[pl.ds(row0, RPT)])

    def wait_s():
        pltpu.make_async_copy(zblk_hbm, ones_v, ssem).wait()

    def body(i, carry):
        @pl.when(i >= DW)
        def _():
            wait_s()

        pltpu.async_copy(ones_v, acc.at[dst_v.at[i]], ssem, add=True)
        return carry

    for half in range(2):
        pltpu.sync_copy(dst_hbm.at[wid, half], dst_v)
        if half == 0:
            plsc.subcore_barrier()
        lax.fori_loop(0, HEB, body, 0)
        for _ in range(DW):
            wait_s()
    plsc.subcore_barrier()
    for k in range(RPT // ZR):
        r = row0 + k * ZR
        pltpu.sync_copy(acc.at[pl.ds(r, ZR)], out_hbm.at[c, pl.ds(r, ZR)])


@functools.partial(
    pl.kernel,
    out_type=jax.ShapeDtypeStruct((NC, NP2, H), jnp.float32),
    mesh=_mesh,
    scratch_types=[
        pltpu.VMEM((HEB, B2), jnp.int32),
        pltpu.VMEM((HEB, B2), jnp.int32),
        pltpu.VMEM((B2, H), jnp.float32),
        pltpu.VMEM((B2, H), jnp.float32),
        pltpu.VMEM_SHARED((NP2, H), jnp.float32),
        pltpu.SemaphoreType.DMA,
    ],
)
def _sc_scatter(table_hbm, src_hbm, dst_hbm, zeros_hbm, zblk_hbm, out_hbm,
                src_v, dst_v, rows0, rows1, acc, gsem):
    c = lax.axis_index("c")
    s = lax.axis_index("s")
    wid = c * NS + s
    row0 = s * RPT
    pltpu.sync_copy(zeros_hbm, acc.at[pl.ds(row0, RPT)])

    def gather(i, buf):
        pltpu.async_copy(table_hbm.at[src_v.at[i]], buf, gsem)

    def wait_g(buf):
        pltpu.make_async_copy(zblk_hbm, buf, gsem).wait()

    def scat(i, buf):
        pltpu.sync_copy(buf, acc.at[dst_v.at[i]], add=True)

    NG2 = HEB // 2
    for half in range(2):
        pltpu.sync_copy(src_hbm.at[wid, half], src_v)
        pltpu.sync_copy(dst_hbm.at[wid, half], dst_v)
        if half == 0:
            plsc.subcore_barrier()
        gather(0, rows0)

        def grp(g, carry):
            i0 = 2 * g
            wait_g(rows0)
            gather(i0 + 1, rows1)
            scat(i0, rows0)

            wait_g(rows1)

            @pl.when(g < NG2 - 1)
            def _():
                gather(i0 + 2, rows0)

            scat(i0 + 1, rows1)
            return carry

        lax.fori_loop(0, NG2, grp, 0)
    plsc.subcore_barrier()
    for k in range(RPT // ZR):
        r = row0 + k * ZR
        pltpu.sync_copy(acc.at[pl.ds(r, ZR)], out_hbm.at[c, pl.ds(r, ZR)])


BN = 1000
NBLK = N // BN


def _dinv(deg_ref):
    d = deg_ref[0, :, 0:1] + deg_ref[1, :, 0:1] + 1.0
    return lax.rsqrt(jnp.maximum(d, 1.0))


def _tc_prologue_body(deg_ref, x_ref, w_ref, out_ref):
    out_ref[...] = jnp.dot(x_ref[...], w_ref[...],
                           preferred_element_type=jnp.float32) * _dinv(deg_ref)


def _combine(deg_ref, s2_ref, hwp_ref, b_ref):
    dinv = _dinv(deg_ref)
    return (s2_ref[0] + s2_ref[1] + hwp_ref[...]) * dinv + b_ref[...], dinv


def _layernorm(t, g_ref, be_ref):
    mu = jnp.mean(t, axis=-1, keepdims=True)
    var = jnp.mean((t - mu) ** 2, axis=-1, keepdims=True)
    return (t - mu) * lax.rsqrt(var + 1e-5) * g_ref[...] + be_ref[...]


def _tc_layer0_body(deg_ref, s2_ref, hwp_ref, b_ref, w_ref, h_ref, hn_ref):
    t, dinv = _combine(deg_ref, s2_ref, hwp_ref, b_ref)
    h = jnp.maximum(t, 0.0)
    h_ref[...] = h
    hn_ref[...] = jnp.dot(h, w_ref[...], preferred_element_type=jnp.float32) * dinv


def _tc_mid_body(deg_ref, s2_ref, hwp_ref, hp_ref, g_ref, be_ref, b_ref, w_ref,
                 h_ref, hn_ref):
    t, dinv = _combine(deg_ref, s2_ref, hwp_ref, b_ref)
    h = jnp.maximum(_layernorm(t, g_ref, be_ref), 0.0) + hp_ref[...]
    h_ref[...] = h
    hn_ref[...] = jnp.dot(h, w_ref[...], preferred_element_type=jnp.float32) * dinv


def _tc_final_body(deg_ref, s2_ref, hwp_ref, hp_ref, g_ref, be_ref, b_ref,
                   batch_ref, fc1w_ref, fc1b_ref, fc2w_ref, fc2b_ref,
                   out_ref, acc, cnt):
    i = pl.program_id(0)
    t, _ = _combine(deg_ref, s2_ref, hwp_ref, b_ref)
    h = jnp.maximum(_layernorm(t, g_ref, be_ref), 0.0) + hp_ref[...]

    bt = batch_ref[0, 0, :]
    onehot = (bt[:, None] == lax.broadcasted_iota(jnp.int32, (1, G), 1)
              ).astype(jnp.float32)
    part = lax.dot_general(onehot, h, (((0,), (0,)), ((), ())),
                           preferred_element_type=jnp.float32)
    cpart = lax.dot_general(onehot, jnp.ones_like(h), (((0,), (0,)), ((), ())),
                            preferred_element_type=jnp.float32)

    @pl.when(i == 0)
    def _():
        acc[...] = jnp.zeros_like(acc)
        cnt[...] = jnp.zeros_like(cnt)

    acc[...] += part
    cnt[...] += cpart

    @pl.when(i == pl.num_programs(0) - 1)
    def _():
        pooled = acc[...] / jnp.maximum(cnt[...], 1.0)
        z = jnp.maximum(jnp.dot(pooled, fc1w_ref[...],
                                preferred_element_type=jnp.float32)
                        + fc1b_ref[...], 0.0)
        o = jnp.dot(z, fc2w_ref[...], preferred_element_type=jnp.float32) \
            + fc2b_ref[...]
        out_ref[...] = 1.0 / (1.0 + jnp.exp(-o))


_deg_spec = pl.BlockSpec((2, BN, H), lambda i: (0, i, 0))
_s2_spec = pl.BlockSpec((2, BN, H), lambda i: (0, i, 0))
_row_spec = pl.BlockSpec((BN, H), lambda i: (i, 0))
_vec_spec = pl.BlockSpec((1, H), lambda i: (0, 0))
_w_spec = pl.BlockSpec((H, H), lambda i: (0, 0))

_rowout = jax.ShapeDtypeStruct((N, H), jnp.float32)

_tc_prologue = pl.pallas_call(
    _tc_prologue_body, grid=(NBLK,),
    in_specs=[_deg_spec, _row_spec, _w_spec],
    out_specs=_row_spec, out_shape=_rowout)

_tc_layer0 = pl.pallas_call(
    _tc_layer0_body, grid=(NBLK,),
    in_specs=[_deg_spec, _s2_spec, _row_spec, _vec_spec, _w_spec],
    out_specs=(_row_spec, _row_spec), out_shape=(_rowout, _rowout))

_tc_mid = pl.pallas_call(
    _tc_mid_body, grid=(NBLK,),
    in_specs=[_deg_spec, _s2_spec, _row_spec, _row_spec, _vec_spec, _vec_spec,
              _vec_spec, _w_spec],
    out_specs=(_row_spec, _row_spec), out_shape=(_rowout, _rowout))

_tc_final = pl.pallas_call(
    _tc_final_body, grid=(NBLK,),
    in_specs=[_deg_spec, _s2_spec, _row_spec, _row_spec, _vec_spec, _vec_spec,
              _vec_spec,
              pl.BlockSpec((1, 1, BN), lambda i: (i, 0, 0)),
              pl.BlockSpec((H, 64), lambda i: (0, 0)),
              pl.BlockSpec((1, 64), lambda i: (0, 0)),
              pl.BlockSpec((64, H), lambda i: (0, 0)),
              _vec_spec],
    out_specs=pl.BlockSpec((G, H), lambda i: (0, 0)),
    out_shape=jax.ShapeDtypeStruct((G, H), jnp.float32),
    scratch_shapes=[pltpu.VMEM((G, H), jnp.float32),
                    pltpu.VMEM((G, H), jnp.float32)])


def kernel(x, edge_index, batch, Ws, bs, gammas, betas, fc1_W, fc1_b, fc2_W,
           fc2_b):
    src = edge_index[0].astype(jnp.int32).reshape(NW, 2, HEB, B2)
    dst = edge_index[1].astype(jnp.int32).reshape(NW, 2, HEB, B2)
    batch3 = batch.astype(jnp.int32).reshape(NBLK, 1, BN)

    ones16 = jnp.ones((B2, H), jnp.float32)
    zerosH = jnp.zeros((RPT, H), jnp.float32)
    zblk = jnp.zeros((B2, H), jnp.float32)

    bs2 = bs.reshape(8, 1, H)
    g2 = gammas.reshape(7, 1, H)
    be2 = betas.reshape(7, 1, H)
    fc1b2 = fc1_b.reshape(1, 64)
    fc2wp = jnp.pad(fc2_W, ((0, 0), (0, H - 1)))
    fc2bp = jnp.pad(fc2_b, (0, H - 1)).reshape(1, H)

    deg2 = _sc_degree(dst, ones16, zerosH, zblk)
    hwp = _tc_prologue(deg2, x, Ws[0])

    h = None
    out = None
    for i in range(8):
        s2 = _sc_scatter(hwp, src, dst, zerosH, zblk)
        if i == 0:
            h, hwp = _tc_layer0(deg2, s2, hwp, bs2[0], Ws[1])
        elif i < 7:
            h, hwp = _tc_mid(deg2, s2, hwp, h, g2[i - 1], be2[i - 1], bs2[i],
                             Ws[i + 1])
        else:
            out = _tc_final(deg2, s2, hwp, h, g2[6], be2[6], bs2[7], batch3,
                            fc1_W, fc1b2, fc2wp, fc2bp)
    return out[:, :1]


# speedup vs baseline: 3.3013x; 1.0650x over previous
import functools

import jax
import jax.numpy as jnp
from jax import lax
from jax.experimental import pallas as pl
from jax.experimental.pallas import tpu as pltpu
from jax.experimental.pallas import tpu_sc as plsc

N = 10000
E = 320000
H = 128
G = 16

NC = 2
NS = 16
NW = NC * NS
B = 80
EPW = E // NW
EB = EPW // B
B2 = 125
HEB = 40
DW = 4
NP2 = 10240
RPT = NP2 // NS
ZR = 128

_mesh = plsc.VectorSubcoreMesh(core_axis_name="c", subcore_axis_name="s")


@functools.partial(
    pl.kernel,
    out_type=jax.ShapeDtypeStruct((NC, NP2, H), jnp.float32),
    mesh=_mesh,
    scratch_types=[
        pltpu.VMEM((HEB, B2), jnp.int32),
        pltpu.VMEM((B2, H), jnp.float32),
        pltpu.VMEM_SHARED((NP2, H), jnp.float32),
        pltpu.SemaphoreType.DMA,
    ],
)
def _sc_degree(dst_hbm, ones_hbm, zeros_hbm, zblk_hbm, out_hbm,
               dst_v, ones_v, acc, ssem):
    c = lax.axis_index("c")
    s = lax.axis_index("s")
    wid = c * NS + s
    pltpu.sync_copy(ones_hbm, ones_v)
    row0 = s * RPT
    pltpu.sync_copy(zeros_hbm, acc.at[pl.ds(row0, RPT)])

    def wait_s():
        pltpu.make_async_copy(zblk_hbm, ones_v, ssem).wait()

    def body(i, carry):
        @pl.when(i >= DW)
        def _():
            wait_s()

        pltpu.async_copy(ones_v, acc.at[dst_v.at[i]], ssem, add=True)
        return carry

    for half in range(2):
        pltpu.sync_copy(dst_hbm.at[wid, half], dst_v)
        if half == 0:
            plsc.subcore_barrier()
        lax.fori_loop(0, HEB, body, 0)
        for _ in range(DW):
            wait_s()
    plsc.subcore_barrier()
    for k in range(RPT // ZR):
        r = row0 + k * ZR
        pltpu.sync_copy(acc.at[pl.ds(r, ZR)], out_hbm.at[c, pl.ds(r, ZR)])


@functools.partial(
    pl.kernel,
    out_type=jax.ShapeDtypeStruct((NC, NP2, H), jnp.float32),
    mesh=_mesh,
    scratch_types=[
        pltpu.VMEM((HEB, B2), jnp.int32),
        pltpu.VMEM((HEB, B2), jnp.int32),
        pltpu.VMEM((B2, H), jnp.float32),
        pltpu.VMEM((B2, H), jnp.float32),
        pltpu.VMEM_SHARED((NP2, H), jnp.float32),
        pltpu.SemaphoreType.DMA,
    ],
)
def _sc_scatter(table_hbm, src_hbm, dst_hbm, zeros_hbm, zblk_hbm, out_hbm,
                src_v, dst_v, rows0, rows1, acc, gsem):
    c = lax.axis_index("c")
    s = lax.axis_index("s")
    wid = c * NS + s
    row0 = s * RPT
    pltpu.sync_copy(zeros_hbm, acc.at[pl.ds(row0, RPT)])

    def gather(i, buf):
        pltpu.async_copy(table_hbm.at[src_v.at[i]], buf, gsem)

    def wait_g(buf):
        pltpu.make_async_copy(zblk_hbm, buf, gsem).wait()

    def scat(i, buf):
        pltpu.sync_copy(buf, acc.at[dst_v.at[i]], add=True)

    NG2 = HEB // 2
    for half in range(2):
        pltpu.sync_copy(src_hbm.at[wid, half], src_v)
        pltpu.sync_copy(dst_hbm.at[wid, half], dst_v)
        if half == 0:
            plsc.subcore_barrier()
        gather(0, rows0)

        def grp(g, carry):
            i0 = 2 * g
            wait_g(rows0)
            gather(i0 + 1, rows1)
            scat(i0, rows0)

            wait_g(rows1)

            @pl.when(g < NG2 - 1)
            def _():
                gather(i0 + 2, rows0)

            scat(i0 + 1, rows1)
            return carry

        lax.fori_loop(0, NG2, grp, 0)
    plsc.subcore_barrier()
    for k in range(RPT // ZR):
        r = row0 + k * ZR
        pltpu.sync_copy(acc.at[pl.ds(r, ZR)], out_hbm.at[c, pl.ds(r, ZR)])


BN = 1000
NBLK = N // BN


def _dinv(deg_ref):
    d = deg_ref[0, :, 0:1] + deg_ref[1, :, 0:1] + 1.0
    return lax.rsqrt(jnp.maximum(d, 1.0))


def _tc_prologue_body(deg_ref, x_ref, w_ref, out_ref):
    out_ref[...] = jnp.dot(x_ref[...], w_ref[...],
                           preferred_element_type=jnp.float32) * _dinv(deg_ref)


def _combine(deg_ref, s2_ref, hwp_ref, b_ref):
    dinv = _dinv(deg_ref)
    return (s2_ref[0] + s2_ref[1] + hwp_ref[...]) * dinv + b_ref[...], dinv


def _layernorm(t, g_ref, be_ref):
    mu = jnp.mean(t, axis=-1, keepdims=True)
    var = jnp.mean((t - mu) ** 2, axis=-1, keepdims=True)
    return (t - mu) * lax.rsqrt(var + 1e-5) * g_ref[...] + be_ref[...]


def _tc_layer0_body(deg_ref, s2_ref, hwp_ref, b_ref, w_ref, h_ref, hn_ref):
    t, dinv = _combine(deg_ref, s2_ref, hwp_ref, b_ref)
    h = jnp.maximum(t, 0.0)
    h_ref[...] = h
    hn_ref[...] = jnp.dot(h, w_ref[...], preferred_element_type=jnp.float32) * dinv


def _tc_mid_body(deg_ref, s2_ref, hwp_ref, hp_ref, g_ref, be_ref, b_ref, w_ref,
                 h_ref, hn_ref):
    t, dinv = _combine(deg_ref, s2_ref, hwp_ref, b_ref)
    h = jnp.maximum(_layernorm(t, g_ref, be_ref), 0.0) + hp_ref[...]
    h_ref[...] = h
    hn_ref[...] = jnp.dot(h, w_ref[...], preferred_element_type=jnp.float32) * dinv


def _tc_final_body(deg_ref, s2_ref, hwp_ref, hp_ref, g_ref, be_ref, b_ref,
                   batch_ref, fc1w_ref, fc1b_ref, fc2w_ref, fc2b_ref,
                   out_ref, acc, cnt):
    i = pl.program_id(0)
    t, _ = _combine(deg_ref, s2_ref, hwp_ref, b_ref)
    h = jnp.maximum(_layernorm(t, g_ref, be_ref), 0.0) + hp_ref[...]

    bt = batch_ref[0, 0, :]
    onehot = (bt[:, None] == lax.broadcasted_iota(jnp.int32, (1, G), 1)
              ).astype(jnp.float32)
    part = lax.dot_general(onehot, h, (((0,), (0,)), ((), ())),
                           preferred_element_type=jnp.float32)
    cpart = lax.dot_general(onehot, jnp.ones_like(h), (((0,), (0,)), ((), ())),
                            preferred_element_type=jnp.float32)

    @pl.when(i == 0)
    def _():
        acc[...] = jnp.zeros_like(acc)
        cnt[...] = jnp.zeros_like(cnt)

    acc[...] += part
    cnt[...] += cpart

    @pl.when(i == pl.num_programs(0) - 1)
    def _():
        pooled = acc[...] / jnp.maximum(cnt[...], 1.0)
        z = jnp.maximum(jnp.dot(pooled, fc1w_ref[...],
                                preferred_element_type=jnp.float32)
                        + fc1b_ref[...], 0.0)
        o = jnp.dot(z, fc2w_ref[...], preferred_element_type=jnp.float32) \
            + fc2b_ref[...]
        out_ref[...] = 1.0 / (1.0 + jnp.exp(-o))


_deg_spec = pl.BlockSpec((2, BN, H), lambda i: (0, i, 0))
_s2_spec = pl.BlockSpec((2, BN, H), lambda i: (0, i, 0))
_row_spec = pl.BlockSpec((BN, H), lambda i: (i, 0))
_vec_spec = pl.BlockSpec((1, H), lambda i: (0, 0))
_w_spec = pl.BlockSpec((H, H), lambda i: (0, 0))

_rowout = jax.ShapeDtypeStruct((N, H), jnp.float32)

_tc_prologue = pl.pallas_call(
    _tc_prologue_body, grid=(NBLK,),
    in_specs=[_deg_spec, _row_spec, _w_spec],
    out_specs=_row_spec, out_shape=_rowout)

_tc_layer0 = pl.pallas_call(
    _tc_layer0_body, grid=(NBLK,),
    in_specs=[_deg_spec, _s2_spec, _row_spec, _vec_spec, _w_spec],
    out_specs=(_row_spec, _row_spec), out_shape=(_rowout, _rowout))

_tc_mid = pl.pallas_call(
    _tc_mid_body, grid=(NBLK,),
    in_specs=[_deg_spec, _s2_spec, _row_spec, _row_spec, _vec_spec, _vec_spec,
              _vec_spec, _w_spec],
    out_specs=(_row_spec, _row_spec), out_shape=(_rowout, _rowout))

_tc_final = pl.pallas_call(
    _tc_final_body, grid=(NBLK,),
    in_specs=[_deg_spec, _s2_spec, _row_spec, _row_spec, _vec_spec, _vec_spec,
              _vec_spec,
              pl.BlockSpec((1, 1, BN), lambda i: (i, 0, 0)),
              pl.BlockSpec((H, 64), lambda i: (0, 0)),
              pl.BlockSpec((1, 64), lambda i: (0, 0)),
              pl.BlockSpec((64, H), lambda i: (0, 0)),
              _vec_spec],
    out_specs=pl.BlockSpec((G, H), lambda i: (0, 0)),
    out_shape=jax.ShapeDtypeStruct((G, H), jnp.float32),
    scratch_shapes=[pltpu.VMEM((G, H), jnp.float32),
                    pltpu.VMEM((G, H), jnp.float32)])


def kernel(x, edge_index, batch, Ws, bs, gammas, betas, fc1_W, fc1_b, fc2_W,
           fc2_b):
    src = edge_index[0].astype(jnp.int32).reshape(NW, 2, HEB, B2)
    dst = edge_index[1].astype(jnp.int32).reshape(NW, 2, HEB, B2)
    batch3 = batch.astype(jnp.int32).reshape(NBLK, 1, BN)

    ones16 = jnp.ones((B2, H), jnp.float32)
    zerosH = jnp.zeros((RPT, H), jnp.float32)
    zblk = jnp.zeros((B2, H), jnp.float32)

    bs2 = bs.reshape(8, 1, H)
    g2 = gammas.reshape(7, 1, H)
    be2 = betas.reshape(7, 1, H)
    fc1b2 = fc1_b.reshape(1, 64)
    fc2wp = jnp.pad(fc2_W, ((0, 0), (0, H - 1)))
    fc2bp = jnp.pad(fc2_b, (0, H - 1)).reshape(1, H)

    deg2 = _sc_degree(dst, ones16, zerosH, zblk)
    hwp = _tc_prologue(deg2, x, Ws[0])

    h = None
    out = None
    for i in range(8):
        s2 = _sc_scatter(hwp, src, dst, zerosH, zblk)
        if i == 0:
            h, hwp = _tc_layer0(deg2, s2, hwp, bs2[0], Ws[1])
        elif i < 7:
            h, hwp = _tc_mid(deg2, s2, hwp, h, g2[i - 1], be2[i - 1], bs2[i],
                             Ws[i + 1])
        else:
            out = _tc_final(deg2, s2, hwp, h, g2[6], be2[6], bs2[7], batch3,
                            fc1_W, fc1b2, fc2wp, fc2bp)
    return out[:, :1]
